# Initial kernel scaffold; baseline (speedup 1.0000x reference)
import jax
import jax.numpy as jnp
from jax.experimental import pallas as pl


def kernel(message, attrs, emb0, emb1, emb2):
    raise NotImplementedError("write your pallas kernel here")


# speedup vs baseline: 3.2617x; 3.2617x over previous
import functools

import jax
import jax.numpy as jnp
from jax import lax
from jax.experimental import pallas as pl
from jax.experimental.pallas import tpu as pltpu
from jax.experimental.pallas import tpu_sc as plsc

E = 320000
DIM = 128
V0, V1, V2 = 5, 6, 2
NT = V0 * V1 * V2
NC, NS, L = 2, 16, 16
NW = NC * NS
PER_W = E // NW
CH = 400
NCHUNK = PER_W // CH


def kernel(message, attrs, emb0, emb1, emb2):
    a = attrs.astype(jnp.int32)
    a0, a1, a2 = a[:, 0], a[:, 1], a[:, 2]
    mesh = plsc.VectorSubcoreMesh(core_axis_name="c", subcore_axis_name="s")

    @functools.partial(
        pl.kernel,
        out_type=jax.ShapeDtypeStruct((E, DIM), jnp.float32),
        mesh=mesh,
        scratch_types=[
            pltpu.VMEM((NT, DIM), jnp.float32),
            pltpu.VMEM((V0, DIM), jnp.float32),
            pltpu.VMEM((V1, DIM), jnp.float32),
            pltpu.VMEM((V2, DIM), jnp.float32),
            pltpu.VMEM((CH, DIM), jnp.float32),
            pltpu.VMEM((CH,), jnp.int32),
            pltpu.VMEM((CH,), jnp.int32),
            pltpu.VMEM((CH,), jnp.int32),
            pltpu.VMEM((CH,), jnp.int32),
        ],
    )
    def k(msg_hbm, a0_hbm, a1_hbm, a2_hbm, e0_hbm, e1_hbm, e2_hbm, out_hbm,
          tbl, e0v, e1v, e2v, mbuf, i0, i1, i2, cbuf):
        wid = lax.axis_index("s") * NC + lax.axis_index("c")
        base = wid * PER_W

        pltpu.sync_copy(e0_hbm, e0v)
        pltpu.sync_copy(e1_hbm, e1v)
        pltpu.sync_copy(e2_hbm, e2v)

        def build(cc, carry):
            k0 = cc % V0
            k1 = (cc // V0) % V1
            k2 = cc // (V0 * V1)
            for j in range(DIM // L):
                sl = pl.ds(j * L, L)
                tbl[cc, sl] = e0v[k0, sl] + e1v[k1, sl] + e2v[k2, sl]
            return carry

        lax.fori_loop(0, NT, build, 0)

        def chunk_body(t, carry):
            row0 = base + t * CH
            rows = pl.ds(row0, CH)
            pltpu.sync_copy(msg_hbm.at[rows], mbuf)
            pltpu.sync_copy(a0_hbm.at[rows], i0)
            pltpu.sync_copy(a1_hbm.at[rows], i1)
            pltpu.sync_copy(a2_hbm.at[rows], i2)

            def fuse(g, c2):
                sl = pl.ds(g * L, L)
                cbuf[sl] = i0[sl] + V0 * i1[sl] + (V0 * V1) * i2[sl]
                return c2

            lax.fori_loop(0, CH // L, fuse, 0)

            def grp_body(g, c2):
                civ = cbuf[pl.ds(g * L, L)]
                for lane in range(L):
                    c = civ[lane]
                    e = g * L + lane
                    for j in range(DIM // L):
                        sl = pl.ds(j * L, L)
                        mbuf[e, sl] = jnp.maximum(mbuf[e, sl] + tbl[c, sl], 0.0)
                return c2

            lax.fori_loop(0, CH // L, grp_body, 0)
            pltpu.sync_copy(mbuf, out_hbm.at[rows])
            return carry

        lax.fori_loop(0, NCHUNK, chunk_body, 0)

    return k(message, a0, a1, a2, emb0, emb1, emb2)


# speedup vs baseline: 4.0890x; 1.2536x over previous
import functools

import jax
import jax.numpy as jnp
from jax import lax
from jax.experimental import pallas as pl
from jax.experimental.pallas import tpu as pltpu
from jax.experimental.pallas import tpu_sc as plsc

E = 320000
DIM = 128
V0, V1, V2 = 5, 6, 2
NT = V0 * V1 * V2
NC, NS, L = 2, 16, 16
NW = NC * NS
PER_W = E // NW
CH = 80
NCHUNK = PER_W // CH
NPAIR = (NCHUNK - 1) // 2


def kernel(message, attrs, emb0, emb1, emb2):
    a = attrs.astype(jnp.int32)
    a0, a1, a2 = a[:, 0], a[:, 1], a[:, 2]
    mesh = plsc.VectorSubcoreMesh(core_axis_name="c", subcore_axis_name="s")

    @functools.partial(
        pl.kernel,
        out_type=jax.ShapeDtypeStruct((E, DIM), jnp.float32),
        mesh=mesh,
        scratch_types=[
            pltpu.VMEM((NT, DIM), jnp.float32),
            pltpu.VMEM((V0, DIM), jnp.float32),
            pltpu.VMEM((V1, DIM), jnp.float32),
            pltpu.VMEM((V2, DIM), jnp.float32),
            pltpu.VMEM((CH, DIM), jnp.float32),
            pltpu.VMEM((CH, DIM), jnp.float32),
            pltpu.VMEM((CH, DIM), jnp.float32),
            pltpu.VMEM((CH, DIM), jnp.float32),
            pltpu.VMEM((CH,), jnp.int32),
            pltpu.VMEM((CH,), jnp.int32),
            pltpu.VMEM((CH,), jnp.int32),
            pltpu.VMEM((CH,), jnp.int32),
            pltpu.VMEM((CH,), jnp.int32),
            pltpu.VMEM((CH,), jnp.int32),
            pltpu.SemaphoreType.DMA,
            pltpu.SemaphoreType.DMA,
            pltpu.SemaphoreType.DMA,
            pltpu.SemaphoreType.DMA,
        ],
    )
    def k(msg_hbm, a0_hbm, a1_hbm, a2_hbm, e0_hbm, e1_hbm, e2_hbm, out_hbm,
          tbl, e0v, e1v, e2v, mbuf0, mbuf1, obuf0, obuf1,
          i00, i10, i20, i01, i11, i21,
          insem0, insem1, outsem0, outsem1):
        mbufs = (mbuf0, mbuf1)
        obufs = (obuf0, obuf1)
        ibufs = ((i00, i10, i20), (i01, i11, i21))
        insems = (insem0, insem1)
        outsems = (outsem0, outsem1)

        wid = lax.axis_index("s") * NC + lax.axis_index("c")
        base = wid * PER_W

        pltpu.sync_copy(e0_hbm, e0v)
        pltpu.sync_copy(e1_hbm, e1v)
        pltpu.sync_copy(e2_hbm, e2v)

        def build(cc, carry):
            k0 = cc % V0
            k1 = (cc // V0) % V1
            k2 = cc // (V0 * V1)
            for j in range(DIM // L):
                sl = pl.ds(j * L, L)
                tbl[cc, sl] = e0v[k0, sl] + e1v[k1, sl] + e2v[k2, sl]
            return carry

        lax.fori_loop(0, NT, build, 0)

        def start_in(t, b):
            rows = pl.ds(base + t * CH, CH)
            pltpu.async_copy(msg_hbm.at[rows], mbufs[b], insems[b])
            pltpu.async_copy(a0_hbm.at[rows], ibufs[b][0], insems[b])
            pltpu.async_copy(a1_hbm.at[rows], ibufs[b][1], insems[b])
            pltpu.async_copy(a2_hbm.at[rows], ibufs[b][2], insems[b])

        def wait_in(t, b):
            rows = pl.ds(base + t * CH, CH)
            pltpu.make_async_copy(msg_hbm.at[rows], mbufs[b], insems[b]).wait()
            pltpu.make_async_copy(a0_hbm.at[rows], ibufs[b][0], insems[b]).wait()
            pltpu.make_async_copy(a1_hbm.at[rows], ibufs[b][1], insems[b]).wait()
            pltpu.make_async_copy(a2_hbm.at[rows], ibufs[b][2], insems[b]).wait()

        def start_out(t, b):
            rows = pl.ds(base + t * CH, CH)
            pltpu.async_copy(obufs[b], out_hbm.at[rows], outsems[b])

        def wait_out(t, b):
            rows = pl.ds(base + t * CH, CH)
            pltpu.make_async_copy(obufs[b], out_hbm.at[rows], outsems[b]).wait()

        def compute(b):
            mb, ob = mbufs[b], obufs[b]
            i0b, i1b, i2b = ibufs[b]

            def grp(g, c2):
                sl16 = pl.ds(g * L, L)
                civ = i0b[sl16] + V0 * i1b[sl16] + (V0 * V1) * i2b[sl16]
                for lane in range(L):
                    c = civ[lane]
                    e = g * L + lane
                    for j in range(DIM // L):
                        sl = pl.ds(j * L, L)
                        ob[e, sl] = jnp.maximum(mb[e, sl] + tbl[c, sl], 0.0)
                return c2

            lax.fori_loop(0, CH // L, grp, 0)

        start_in(0, 0)
        start_in(1, 1)

        def pair_body(p, carry):
            t0 = 2 * p
            wait_in(t0, 0)

            @pl.when(p >= 1)
            def _():
                wait_out(t0 - 2, 0)

            compute(0)
            start_out(t0, 0)
            start_in(t0 + 2, 0)
            wait_in(t0 + 1, 1)

            @pl.when(p >= 1)
            def _():
                wait_out(t0 - 1, 1)

            compute(1)
            start_out(t0 + 1, 1)

            @pl.when(p <= NPAIR - 2)
            def _():
                start_in(t0 + 3, 1)

            return carry

        lax.fori_loop(0, NPAIR, pair_body, 0)

        t_last = NCHUNK - 1
        wait_in(t_last, 0)
        wait_out(t_last - 2, 0)
        compute(0)
        start_out(t_last, 0)
        wait_out(t_last - 1, 1)
        wait_out(t_last, 0)

    return k(message, a0, a1, a2, emb0, emb1, emb2)


# speedup vs baseline: 11.1348x; 2.7231x over previous
import functools

import jax
import jax.numpy as jnp
from jax import lax
from jax.experimental import pallas as pl
from jax.experimental.pallas import tpu as pltpu
from jax.experimental.pallas import tpu_sc as plsc

E = 320000
DIM = 128
V0, V1, V2 = 5, 6, 2
NT = V0 * V1 * V2
NC, NS, L = 2, 16, 16
NW = NC * NS
PER_W = E // NW
CH = 80
NCHUNK = PER_W // CH
NPAIR = (NCHUNK - 1) // 2


def kernel(message, attrs, emb0, emb1, emb2):
    a = attrs.astype(jnp.int32)
    a0, a1, a2 = a[:, 0], a[:, 1], a[:, 2]
    mesh = plsc.VectorSubcoreMesh(core_axis_name="c", subcore_axis_name="s")

    @functools.partial(
        pl.kernel,
        out_type=jax.ShapeDtypeStruct((E, DIM), jnp.float32),
        mesh=mesh,
        scratch_types=[
            pltpu.VMEM((NT, DIM), jnp.float32),
            pltpu.VMEM((V0, DIM), jnp.float32),
            pltpu.VMEM((V1, DIM), jnp.float32),
            pltpu.VMEM((V2, DIM), jnp.float32),
            pltpu.VMEM((CH, DIM), jnp.float32),
            pltpu.VMEM((CH, DIM), jnp.float32),
            pltpu.VMEM((CH, DIM), jnp.float32),
            pltpu.VMEM((CH, DIM), jnp.float32),
            pltpu.VMEM((CH,), jnp.int32),
            pltpu.VMEM((CH,), jnp.int32),
            pltpu.VMEM((CH,), jnp.int32),
            pltpu.VMEM((CH,), jnp.int32),
            pltpu.VMEM((CH,), jnp.int32),
            pltpu.VMEM((CH,), jnp.int32),
            pltpu.SemaphoreType.DMA,
            pltpu.SemaphoreType.DMA,
            pltpu.SemaphoreType.DMA,
            pltpu.SemaphoreType.DMA,
        ],
    )
    def k(msg_hbm, a0_hbm, a1_hbm, a2_hbm, e0_hbm, e1_hbm, e2_hbm, out_hbm,
          tbl, e0v, e1v, e2v, mbuf0, mbuf1, obuf0, obuf1,
          i00, i10, i20, i01, i11, i21,
          insem0, insem1, outsem0, outsem1):
        mbufs = (mbuf0, mbuf1)
        obufs = (obuf0, obuf1)
        ibufs = ((i00, i10, i20), (i01, i11, i21))
        insems = (insem0, insem1)
        outsems = (outsem0, outsem1)

        wid = lax.axis_index("s") * NC + lax.axis_index("c")
        base = wid * PER_W

        pltpu.sync_copy(e0_hbm, e0v)
        pltpu.sync_copy(e1_hbm, e1v)
        pltpu.sync_copy(e2_hbm, e2v)

        def build(cc, carry):
            k0 = cc % V0
            k1 = (cc // V0) % V1
            k2 = cc // (V0 * V1)
            for j in range(DIM // L):
                sl = pl.ds(j * L, L)
                tbl[cc, sl] = e0v[k0, sl] + e1v[k1, sl] + e2v[k2, sl]
            return carry

        lax.fori_loop(0, NT, build, 0)

        def start_in(t, b):
            rows = pl.ds(base + t * CH, CH)
            pltpu.async_copy(msg_hbm.at[rows], mbufs[b], insems[b])
            pltpu.async_copy(a0_hbm.at[rows], ibufs[b][0], insems[b])
            pltpu.async_copy(a1_hbm.at[rows], ibufs[b][1], insems[b])
            pltpu.async_copy(a2_hbm.at[rows], ibufs[b][2], insems[b])

        def wait_in(t, b):
            rows = pl.ds(base + t * CH, CH)
            pltpu.make_async_copy(msg_hbm.at[rows], mbufs[b], insems[b]).wait()
            pltpu.make_async_copy(a0_hbm.at[rows], ibufs[b][0], insems[b]).wait()
            pltpu.make_async_copy(a1_hbm.at[rows], ibufs[b][1], insems[b]).wait()
            pltpu.make_async_copy(a2_hbm.at[rows], ibufs[b][2], insems[b]).wait()

        def start_out(t, b):
            rows = pl.ds(base + t * CH, CH)
            pltpu.async_copy(obufs[b], out_hbm.at[rows], outsems[b])

        def wait_out(t, b):
            rows = pl.ds(base + t * CH, CH)
            pltpu.make_async_copy(obufs[b], out_hbm.at[rows], outsems[b]).wait()

        def compute(b):
            mb, ob = mbufs[b], obufs[b]
            i0b, i1b, i2b = ibufs[b]

            def grp(g, c2):
                sl16 = pl.ds(g * L, L)
                civ = i0b[sl16] + V0 * i1b[sl16] + (V0 * V1) * i2b[sl16]
                for lane in range(L):
                    c = civ[lane]
                    e = g * L + lane
                    ms = [mb[e, pl.ds(j * L, L)] for j in range(DIM // L)]
                    ts = [tbl[c, pl.ds(j * L, L)] for j in range(DIM // L)]
                    for j in range(DIM // L):
                        ob[e, pl.ds(j * L, L)] = jnp.maximum(ms[j] + ts[j], 0.0)
                return c2

            lax.fori_loop(0, CH // L, grp, 0)

        start_in(0, 0)
        start_in(1, 1)

        def pair_body(p, carry):
            t0 = 2 * p
            wait_in(t0, 0)

            @pl.when(p >= 1)
            def _():
                wait_out(t0 - 2, 0)

            compute(0)
            start_out(t0, 0)
            start_in(t0 + 2, 0)
            wait_in(t0 + 1, 1)

            @pl.when(p >= 1)
            def _():
                wait_out(t0 - 1, 1)

            compute(1)
            start_out(t0 + 1, 1)

            @pl.when(p <= NPAIR - 2)
            def _():
                start_in(t0 + 3, 1)

            return carry

        lax.fori_loop(0, NPAIR, pair_body, 0)

        t_last = NCHUNK - 1
        wait_in(t_last, 0)
        wait_out(t_last - 2, 0)
        compute(0)
        start_out(t_last, 0)
        wait_out(t_last - 1, 1)
        wait_out(t_last, 0)

    return k(message, a0, a1, a2, emb0, emb1, emb2)
